# baseline (speedup 1.0000x reference)
import functools

import jax
import jax.numpy as jnp
from jax import lax
from jax.experimental import pallas as pl
from jax.experimental.pallas import tpu as pltpu
from jax.experimental.pallas import tpu_sc as plsc

_N, _C, _K = 65536, 32, 16
_CK = _C * _K
_NC, _NS, _L = 2, 16, 16
_NW = _NC * _NS
_W = 128
_NR = _N * _C // _W
_ROWS_W = _NR // _NW
_R = 128
_NCH = _ROWS_W // _R


def _sc_body(x_hbm, xp_hbm, yp_hbm, sl_hbm, out_hbm,
             xin0, xin1, out0, out1, xpv, ypv, slv,
             si0, si1, so0, so1):
    wid = lax.axis_index("s") * _NC + lax.axis_index("c")
    row_w = wid * _ROWS_W

    pltpu.sync_copy(xp_hbm, xpv)
    pltpu.sync_copy(yp_hbm, ypv)
    pltpu.sync_copy(sl_hbm, slv)

    lane = lax.iota(jnp.int32, 16)
    base_even = lane * _K
    base_odd = base_even + 16 * _K

    xins = (xin0, xin1)
    outs = (out0, out1)
    sis = (si0, si1)
    sos = (so0, so1)

    in_cp = [
        pltpu.async_copy(x_hbm.at[pl.ds(row_w, _R)], xin0, si0),
        pltpu.async_copy(x_hbm.at[pl.ds(row_w + _R, _R)], xin1, si1),
    ]
    out_cp = [None, None]

    for ch in range(_NCH):
        b = ch % 2
        xin = xins[b]
        outv = outs[b]
        in_cp[b].wait()
        if out_cp[b] is not None:
            out_cp[b].wait()

        def body(i, _, xin=xin, outv=outv):
            for k in range(8):
                cbase = base_even if k % 2 == 0 else base_odd
                xv = xin[i, pl.ds(16 * k, 16)]
                t = jnp.minimum(jnp.maximum(xv * 7.5 + 7.5, 0.0), 14.0)
                idx = cbase + t.astype(jnp.int32)
                xpj = plsc.load_gather(xpv, [idx])
                y0 = plsc.load_gather(ypv, [idx])
                s = plsc.load_gather(slv, [idx])
                outv[i, pl.ds(16 * k, 16)] = y0 + (xv - xpj) * s
            return 0

        lax.fori_loop(0, _R, body, 0)

        out_cp[b] = pltpu.async_copy(
            outv, out_hbm.at[pl.ds(row_w + ch * _R, _R)], sos[b])
        nxt = ch + 2
        if nxt < _NCH:
            in_cp[b] = pltpu.async_copy(
                x_hbm.at[pl.ds(row_w + nxt * _R, _R)], xins[b], sis[b])

    out_cp[0].wait()
    out_cp[1].wait()


_pwl_call = functools.partial(
    pl.kernel,
    mesh=plsc.VectorSubcoreMesh(core_axis_name="c", subcore_axis_name="s"),
    out_type=jax.ShapeDtypeStruct((_NR, _W), jnp.float32),
    compiler_params=pltpu.CompilerParams(
        needs_layout_passes=False, use_tc_tiling_on_sc=True),
    scratch_types=[
        pltpu.VMEM((_R, _W), jnp.float32),
        pltpu.VMEM((_R, _W), jnp.float32),
        pltpu.VMEM((_R, _W), jnp.float32),
        pltpu.VMEM((_R, _W), jnp.float32),
        pltpu.VMEM((_CK,), jnp.float32),
        pltpu.VMEM((_CK,), jnp.float32),
        pltpu.VMEM((_CK,), jnp.float32),
        pltpu.SemaphoreType.DMA,
        pltpu.SemaphoreType.DMA,
        pltpu.SemaphoreType.DMA,
        pltpu.SemaphoreType.DMA,
    ],
)(_sc_body)


def kernel(x, xp, yp):
    n, c = x.shape
    slope = (yp[:, 1:] - yp[:, :-1]) / (xp[:, 1:] - xp[:, :-1] + 1e-7)
    slope = jnp.concatenate([slope, jnp.zeros((c, 1), jnp.float32)], axis=1)
    out = _pwl_call(x.reshape(_NR, _W), xp.reshape(-1), yp.reshape(-1),
                    slope.reshape(-1))
    return out.reshape(n, c)


# speedup vs baseline: 1.4073x; 1.4073x over previous
import functools

import jax
import jax.numpy as jnp
from jax import lax
from jax.experimental import pallas as pl
from jax.experimental.pallas import tpu as pltpu
from jax.experimental.pallas import tpu_sc as plsc

_N, _C, _K = 65536, 32, 16
_CK = _C * _K
_NC, _NS, _L = 2, 16, 16
_NW = _NC * _NS
_ROWS_W = _N // _NW
_R = 128
_NCH = _ROWS_W // _R


def _sc_body(x_hbm, xp_hbm, yp_hbm, sl_hbm, out_hbm,
             xin0, xin1, out0, out1, xpv, ypv, slv,
             si0, si1, so0, so1):
    wid = lax.axis_index("s") * _NC + lax.axis_index("c")
    row_w = wid * _ROWS_W

    pltpu.sync_copy(xp_hbm, xpv)
    pltpu.sync_copy(yp_hbm, ypv)
    pltpu.sync_copy(sl_hbm, slv)

    lane = lax.iota(jnp.int32, 16)
    base_even = lane * _K
    base_odd = base_even + 16 * _K

    xins = (xin0, xin1)
    outs = (out0, out1)
    sis = (si0, si1)
    sos = (so0, so1)

    in_cp = [
        pltpu.async_copy(x_hbm.at[pl.ds(row_w, _R)], xin0, si0),
        pltpu.async_copy(x_hbm.at[pl.ds(row_w + _R, _R)], xin1, si1),
    ]
    out_cp = [None, None]

    for ch in range(_NCH):
        b = ch % 2
        xin = xins[b]
        outv = outs[b]
        in_cp[b].wait()
        if out_cp[b] is not None:
            out_cp[b].wait()

        @plsc.parallel_loop(0, _R, unroll=4)
        def body(i, xin=xin, outv=outv):
            for col, cbase in ((0, base_even), (16, base_odd)):
                xv = xin[i, pl.ds(col, 16)]
                t = jnp.minimum(jnp.maximum(xv * 7.5 + 7.5, 0.0), 14.0)
                idx = cbase + t.astype(jnp.int32)
                xpj = plsc.load_gather(xpv, [idx])
                y0 = plsc.load_gather(ypv, [idx])
                s = plsc.load_gather(slv, [idx])
                outv[i, pl.ds(col, 16)] = y0 + (xv - xpj) * s

        out_cp[b] = pltpu.async_copy(
            outv, out_hbm.at[pl.ds(row_w + ch * _R, _R)], sos[b])
        nxt = ch + 2
        if nxt < _NCH:
            in_cp[b] = pltpu.async_copy(
                x_hbm.at[pl.ds(row_w + nxt * _R, _R)], xins[b], sis[b])

    out_cp[0].wait()
    out_cp[1].wait()


_pwl_call = functools.partial(
    pl.kernel,
    mesh=plsc.VectorSubcoreMesh(core_axis_name="c", subcore_axis_name="s"),
    out_type=jax.ShapeDtypeStruct((_N, _C), jnp.float32),
    compiler_params=pltpu.CompilerParams(
        needs_layout_passes=False, use_tc_tiling_on_sc=True),
    scratch_types=[
        pltpu.VMEM((_R, _C), jnp.float32),
        pltpu.VMEM((_R, _C), jnp.float32),
        pltpu.VMEM((_R, _C), jnp.float32),
        pltpu.VMEM((_R, _C), jnp.float32),
        pltpu.VMEM((_CK,), jnp.float32),
        pltpu.VMEM((_CK,), jnp.float32),
        pltpu.VMEM((_CK,), jnp.float32),
        pltpu.SemaphoreType.DMA,
        pltpu.SemaphoreType.DMA,
        pltpu.SemaphoreType.DMA,
        pltpu.SemaphoreType.DMA,
    ],
)(_sc_body)


def kernel(x, xp, yp):
    c = x.shape[1]
    slope = (yp[:, 1:] - yp[:, :-1]) / (xp[:, 1:] - xp[:, :-1] + 1e-7)
    slope = jnp.concatenate([slope, jnp.zeros((c, 1), jnp.float32)], axis=1)
    return _pwl_call(x, xp.reshape(-1), yp.reshape(-1), slope.reshape(-1))


# speedup vs baseline: 2.7403x; 1.9472x over previous
import functools

import jax
import jax.numpy as jnp
from jax import lax
from jax.experimental import pallas as pl
from jax.experimental.pallas import tpu as pltpu
from jax.experimental.pallas import tpu_sc as plsc

_N, _C, _K = 65536, 32, 16
_CK = _C * _K
_NC, _NS, _L = 2, 16, 16
_NW = _NC * _NS
_COLS_W = _N // _NW
_WC = 512
_NCH = _COLS_W // _WC
_H = 2.0 / (_K - 1)


def _sc_body(xt_hbm, yp_hbm, sl_hbm, out_hbm,
             xin0, xin1, out0, out1, ypv, slv,
             si0, si1, so0, so1):
    wid = lax.axis_index("s") * _NC + lax.axis_index("c")
    col_w = wid * _COLS_W

    pltpu.sync_copy(yp_hbm, ypv)
    pltpu.sync_copy(sl_hbm, slv)

    xins = (xin0, xin1)
    outs = (out0, out1)
    sis = (si0, si1)
    sos = (so0, so1)

    in_cp = [
        pltpu.async_copy(xt_hbm.at[:, pl.ds(col_w, _WC)], xin0, si0),
        pltpu.async_copy(xt_hbm.at[:, pl.ds(col_w + _WC, _WC)], xin1, si1),
    ]
    out_cp = [None, None]

    for ch in range(_NCH):
        b = ch % 2
        xin = xins[b]
        outv = outs[b]
        in_cp[b].wait()
        if out_cp[b] is not None:
            out_cp[b].wait()

        @plsc.parallel_loop(0, _WC // 16, unroll=1)
        def body(v, xin=xin, outv=outv):
            o = v * 16
            for c in range(_C):
                xv = xin[c, pl.ds(o, 16)]
                t = jnp.minimum(jnp.maximum(xv * 7.5 + 7.5, 0.0), 14.0)
                ji = t.astype(jnp.int32)
                y0 = plsc.load_gather(ypv, [ji + (c * _K)])
                s = plsc.load_gather(slv, [ji + (c * _K)])
                xpj = ji.astype(jnp.float32) * _H - 1.0
                outv[c, pl.ds(o, 16)] = y0 + (xv - xpj) * s

        out_cp[b] = pltpu.async_copy(
            outv, out_hbm.at[:, pl.ds(col_w + ch * _WC, _WC)], sos[b])
        nxt = ch + 2
        if nxt < _NCH:
            in_cp[b] = pltpu.async_copy(
                xt_hbm.at[:, pl.ds(col_w + nxt * _WC, _WC)], xins[b], sis[b])

    out_cp[0].wait()
    out_cp[1].wait()


_pwl_call = functools.partial(
    pl.kernel,
    mesh=plsc.VectorSubcoreMesh(core_axis_name="c", subcore_axis_name="s"),
    out_type=jax.ShapeDtypeStruct((_C, _N), jnp.float32),
    compiler_params=pltpu.CompilerParams(
        needs_layout_passes=False, use_tc_tiling_on_sc=True),
    scratch_types=[
        pltpu.VMEM((_C, _WC), jnp.float32),
        pltpu.VMEM((_C, _WC), jnp.float32),
        pltpu.VMEM((_C, _WC), jnp.float32),
        pltpu.VMEM((_C, _WC), jnp.float32),
        pltpu.VMEM((_CK,), jnp.float32),
        pltpu.VMEM((_CK,), jnp.float32),
        pltpu.SemaphoreType.DMA,
        pltpu.SemaphoreType.DMA,
        pltpu.SemaphoreType.DMA,
        pltpu.SemaphoreType.DMA,
    ],
)(_sc_body)


def kernel(x, xp, yp):
    c = x.shape[1]
    slope = (yp[:, 1:] - yp[:, :-1]) / (xp[:, 1:] - xp[:, :-1] + 1e-7)
    slope = jnp.concatenate([slope, jnp.zeros((c, 1), jnp.float32)], axis=1)
    out_t = _pwl_call(x.T, yp.reshape(-1), slope.reshape(-1))
    return out_t.T
